# baseline (speedup 1.0000x reference)
import functools

import jax
import jax.numpy as jnp
from jax import lax
from jax.experimental import pallas as pl
from jax.experimental.pallas import tpu as pltpu
from jax.experimental.pallas import tpu_sc as plsc

DIM = 1024
B = 16384
NC = 2
NS = 16
NW = NC * NS
B_PER_W = B // NW
CHUNK = 16
NCHUNK = B_PER_W // CHUNK
NBUF = 6


def _make_gather():
    mesh = plsc.VectorSubcoreMesh(core_axis_name="c", subcore_axis_name="s")

    @functools.partial(
        pl.kernel,
        mesh=mesh,
        out_type=jax.ShapeDtypeStruct((B, DIM), jnp.float32),
        scratch_types=[
            pltpu.VMEM((B_PER_W,), jnp.int32),
        ] + [pltpu.VMEM((CHUNK, DIM), jnp.float32) for _ in range(NBUF)]
          + [pltpu.SemaphoreType.DMA for _ in range(2 * NBUF)],
    )
    def gather_kernel(idx_hbm, pe_hbm, out_hbm, idx_v, *scratch):
        bufs = scratch[:NBUF]
        gsems = scratch[NBUF:2 * NBUF]
        wsems = scratch[2 * NBUF:]
        wid = lax.axis_index("s") * NC + lax.axis_index("c")
        base = wid * B_PER_W
        pltpu.sync_copy(idx_hbm.at[pl.ds(base, B_PER_W)], idx_v)

        g_desc = [
            pltpu.async_copy(
                pe_hbm.at[idx_v.at[pl.ds(b * CHUNK, CHUNK)]], bufs[b],
                gsems[b])
            for b in range(NBUF)
        ]
        w_desc = [None] * NBUF
        for j in range(NCHUNK):
            b = j % NBUF
            nj = j + NBUF - 1
            if NBUF <= nj < NCHUNK:
                nb = nj % NBUF
                w_desc[nb].wait()
                g_desc[nb] = pltpu.async_copy(
                    pe_hbm.at[idx_v.at[pl.ds(nj * CHUNK, CHUNK)]], bufs[nb],
                    gsems[nb])
            g_desc[b].wait()
            w_desc[b] = pltpu.async_copy(
                bufs[b], out_hbm.at[pl.ds(base + j * CHUNK, CHUNK)], wsems[b])
        for b in range(NBUF):
            w_desc[b].wait()

    return gather_kernel


_GATHER = _make_gather()


def kernel(t, pe):
    return _GATHER(t.astype(jnp.int32), pe)
